# Initial kernel scaffold; baseline (speedup 1.0000x reference)
import jax
import jax.numpy as jnp
from jax.experimental import pallas as pl


def kernel(V, edge_index, rel_ids, W1, b1, W2, b2):
    raise NotImplementedError("write your pallas kernel here")


# speedup vs baseline: 1.4550x; 1.4550x over previous
import jax
import jax.numpy as jnp
from jax import lax
from jax.experimental import pallas as pl
from jax.experimental.pallas import tpu as pltpu
from jax.experimental.pallas import tpu_sc as plsc

N_NODES = 10000
E_TOT = 320000
D_IN = 128
NUM_REL = 16
HID = 256
HW = 128
HALF = N_NODES // 2
PAD_HALF = 5120
ROWS_PER_TILE = PAD_HALF // 16
N_TILE_EDGES = E_TOT // 16
CHUNK = 128
N_FULL = N_TILE_EDGES // CHUNK
TAIL = N_TILE_EDGES - N_FULL * CHUNK
NB = 10
BLK = N_NODES // NB


def _mm_a_body(v_ref, w1_ref, alo_ref, ahi_ref):
    alo_ref[...] = jnp.dot(v_ref[...], w1_ref[:D_IN, :HW],
                           preferred_element_type=jnp.float32)
    ahi_ref[...] = jnp.dot(v_ref[...], w1_ref[:D_IN, HW:],
                           preferred_element_type=jnp.float32)


def _stage_a(V, W1):
    return pl.pallas_call(
        _mm_a_body,
        grid=(NB,),
        in_specs=[
            pl.BlockSpec((BLK, D_IN), lambda i: (i, 0)),
            pl.BlockSpec((2 * D_IN + NUM_REL, HID), lambda i: (0, 0)),
        ],
        out_specs=[
            pl.BlockSpec((BLK, HW), lambda i: (i, 0)),
            pl.BlockSpec((BLK, HW), lambda i: (i, 0)),
        ],
        out_shape=[
            jax.ShapeDtypeStruct((N_NODES, HW), jnp.float32),
            jax.ShapeDtypeStruct((N_NODES, HW), jnp.float32),
        ],
    )(V, W1)


def _mm_b_body(v_ref, w1_ref, b1_ref, blo_ref, bhi_ref):
    r = pl.program_id(0)
    t = w1_ref[pl.ds(2 * D_IN + r, 1), :] + b1_ref[...]
    mm_lo = jnp.dot(v_ref[...], w1_ref[D_IN:2 * D_IN, :HW],
                    preferred_element_type=jnp.float32)
    mm_hi = jnp.dot(v_ref[...], w1_ref[D_IN:2 * D_IN, HW:],
                    preferred_element_type=jnp.float32)
    blo_ref[0] = mm_lo + t[:, :HW]
    bhi_ref[0] = mm_hi + t[:, HW:]


def _stage_b(V, W1, b1row):
    return pl.pallas_call(
        _mm_b_body,
        grid=(NUM_REL, NB),
        in_specs=[
            pl.BlockSpec((BLK, D_IN), lambda r, i: (i, 0)),
            pl.BlockSpec((2 * D_IN + NUM_REL, HID), lambda r, i: (0, 0)),
            pl.BlockSpec((1, HID), lambda r, i: (0, 0)),
        ],
        out_specs=[
            pl.BlockSpec((1, BLK, HW), lambda r, i: (r, i, 0)),
            pl.BlockSpec((1, BLK, HW), lambda r, i: (r, i, 0)),
        ],
        out_shape=[
            jax.ShapeDtypeStruct((NUM_REL, N_NODES, HW), jnp.float32),
            jax.ShapeDtypeStruct((NUM_REL, N_NODES, HW), jnp.float32),
        ],
    )(V, W1, b1row)


def _zero_stripe_and_barrier(hbuf, h_shared, stripe):
    zero16 = jnp.zeros((16,), jnp.float32)

    def _zbody(e, carry):
        for j in range(HW // 16):
            hbuf[e, pl.ds(16 * j, 16)] = zero16
        return carry

    lax.fori_loop(0, CHUNK, _zbody, 0)
    done = 0
    while done < ROWS_PER_TILE:
        step = min(CHUNK, ROWS_PER_TILE - done)
        h_z = hbuf if step == CHUNK else hbuf.at[pl.ds(0, step)]
        pltpu.sync_copy(h_z, h_shared.at[pl.ds(stripe + done, step)])
        done += step
    plsc.subcore_barrier()


def _copy_out(hbuf, h_shared, h_out_hbm, stripe, out_base):
    plsc.subcore_barrier()
    done = 0
    while done < ROWS_PER_TILE:
        step = min(CHUNK, ROWS_PER_TILE - done)
        h_b = hbuf if step == CHUNK else hbuf.at[pl.ds(0, step)]
        pltpu.sync_copy(h_shared.at[pl.ds(stripe + done, step)], h_b)
        pltpu.sync_copy(h_b, h_out_hbm.at[pl.ds(out_base + done, step)])
        done += step


def _owned_idx(dv, iv, nbase, n_e, iot):
    def _idx_body(j, carry):
        dd = dv[pl.ds(16 * j, 16)]
        owned = (dd >= nbase) & (dd < nbase + HALF)
        iv[pl.ds(16 * j, 16)] = jnp.where(owned, dd - nbase, HALF + iot)
        return carry

    lax.fori_loop(0, n_e // 16, _idx_body, 0)


def _sc_gather_body(a_hbm, b2_hbm, src_hbm, dst_hbm, rel_hbm, h_out_hbm,
                    g1, g2, hbuf, svec, dvec, rvec, cvec, ivec,
                    svec_t, dvec_t, rvec_t, cvec_t, ivec_t, h_shared):
    c = lax.axis_index("c")
    s = lax.axis_index("s")
    nbase = c * HALF
    ebase = s * N_TILE_EDGES
    iot = lax.iota(jnp.int32, 16)
    stripe = s * ROWS_PER_TILE

    _zero_stripe_and_barrier(hbuf, h_shared, stripe)

    def _process(cbase, n_e, sv, dv, rv, cv, iv):
        pltpu.sync_copy(src_hbm.at[pl.ds(cbase, n_e)], sv)
        pltpu.sync_copy(dst_hbm.at[pl.ds(cbase, n_e)], dv)
        pltpu.sync_copy(rel_hbm.at[pl.ds(cbase, n_e)], rv)

        _owned_idx(dv, iv, nbase, n_e, iot)

        def _cvec_body(j, carry):
            rr = rv[pl.ds(16 * j, 16)]
            dd = dv[pl.ds(16 * j, 16)]
            cv[pl.ds(16 * j, 16)] = rr * N_NODES + dd
            return carry

        lax.fori_loop(0, n_e // 16, _cvec_body, 0)

        g1_dst = g1 if n_e == CHUNK else g1.at[pl.ds(0, n_e)]
        g2_dst = g2 if n_e == CHUNK else g2.at[pl.ds(0, n_e)]
        pltpu.sync_copy(a_hbm.at[sv], g1_dst)
        pltpu.sync_copy(b2_hbm.at[cv], g2_dst)

        def _edge_body(e, carry):
            for j in range(HW // 16):
                x = g1[e, pl.ds(16 * j, 16)] + g2[e, pl.ds(16 * j, 16)]
                hbuf[e, pl.ds(16 * j, 16)] = jnp.maximum(x, 0.0)
            return carry

        lax.fori_loop(0, n_e, _edge_body, 0)

        h_src = hbuf if n_e == CHUNK else hbuf.at[pl.ds(0, n_e)]
        pltpu.sync_copy(h_src, h_shared.at[iv], add=True)

    def _chunk_body(i, carry):
        _process(ebase + i * CHUNK, CHUNK, svec, dvec, rvec, cvec, ivec)
        return carry

    lax.fori_loop(0, N_FULL, _chunk_body, 0)
    _process(ebase + N_FULL * CHUNK, TAIL,
             svec_t, dvec_t, rvec_t, cvec_t, ivec_t)

    _copy_out(hbuf, h_shared, h_out_hbm, stripe, c * PAD_HALF + stripe)


def _sc_edges(A, B2, src, dst, rel):
    mesh = plsc.VectorSubcoreMesh(core_axis_name="c", subcore_axis_name="s")
    f = pl.kernel(
        _sc_gather_body,
        mesh=mesh,
        out_type=jax.ShapeDtypeStruct((2 * PAD_HALF, HW), jnp.float32),
        scratch_types=[
            pltpu.VMEM((CHUNK, HW), jnp.float32),
            pltpu.VMEM((CHUNK, HW), jnp.float32),
            pltpu.VMEM((CHUNK, HW), jnp.float32),
            pltpu.VMEM((CHUNK,), jnp.int32),
            pltpu.VMEM((CHUNK,), jnp.int32),
            pltpu.VMEM((CHUNK,), jnp.int32),
            pltpu.VMEM((CHUNK,), jnp.int32),
            pltpu.VMEM((CHUNK,), jnp.int32),
            pltpu.VMEM((TAIL,), jnp.int32),
            pltpu.VMEM((TAIL,), jnp.int32),
            pltpu.VMEM((TAIL,), jnp.int32),
            pltpu.VMEM((TAIL,), jnp.int32),
            pltpu.VMEM((TAIL,), jnp.int32),
            pltpu.VMEM_SHARED((PAD_HALF, HW), jnp.float32),
        ],
    )
    return f(A, B2, src, dst, rel)


def _sc_count_body(dst_hbm, h_out_hbm,
                   hbuf, dvec, ivec, dvec_t, ivec_t, h_shared):
    c = lax.axis_index("c")
    s = lax.axis_index("s")
    nbase = c * HALF
    ebase = s * N_TILE_EDGES
    iot = lax.iota(jnp.int32, 16)
    stripe = s * ROWS_PER_TILE

    _zero_stripe_and_barrier(hbuf, h_shared, stripe)

    one0 = jnp.where(iot == 0, 1.0, 0.0).astype(jnp.float32)

    def _fill_body(e, carry):
        hbuf[e, pl.ds(0, 16)] = one0
        return carry

    lax.fori_loop(0, CHUNK, _fill_body, 0)

    def _process(cbase, n_e, dv, iv):
        pltpu.sync_copy(dst_hbm.at[pl.ds(cbase, n_e)], dv)
        _owned_idx(dv, iv, nbase, n_e, iot)
        h_src = hbuf if n_e == CHUNK else hbuf.at[pl.ds(0, n_e)]
        pltpu.sync_copy(h_src, h_shared.at[iv], add=True)

    def _chunk_body(i, carry):
        _process(ebase + i * CHUNK, CHUNK, dvec, ivec)
        return carry

    lax.fori_loop(0, N_FULL, _chunk_body, 0)
    _process(ebase + N_FULL * CHUNK, TAIL, dvec_t, ivec_t)

    _copy_out(hbuf, h_shared, h_out_hbm, stripe, c * PAD_HALF + stripe)


def _sc_count(dst):
    mesh = plsc.VectorSubcoreMesh(core_axis_name="c", subcore_axis_name="s")
    f = pl.kernel(
        _sc_count_body,
        mesh=mesh,
        out_type=jax.ShapeDtypeStruct((2 * PAD_HALF, HW), jnp.float32),
        scratch_types=[
            pltpu.VMEM((CHUNK, HW), jnp.float32),
            pltpu.VMEM((CHUNK,), jnp.int32),
            pltpu.VMEM((CHUNK,), jnp.int32),
            pltpu.VMEM((TAIL,), jnp.int32),
            pltpu.VMEM((TAIL,), jnp.int32),
            pltpu.VMEM_SHARED((PAD_HALF, HW), jnp.float32),
        ],
    )
    return f(dst)


def _mm_out_body(v_ref, h1_ref, h2_ref, hc_ref, w2_ref, b2_ref, out_ref):
    mm = jnp.dot(h1_ref[...], w2_ref[:HW, :],
                 preferred_element_type=jnp.float32)
    mm += jnp.dot(h2_ref[...], w2_ref[HW:, :],
                  preferred_element_type=jnp.float32)
    cnt = hc_ref[:, 0:1]
    out_ref[...] = v_ref[...] + mm + cnt * b2_ref[...]


def _stage_out(V, H1, H2, HC, W2, b2row):
    return pl.pallas_call(
        _mm_out_body,
        grid=(NB,),
        in_specs=[
            pl.BlockSpec((BLK, D_IN), lambda i: (i, 0)),
            pl.BlockSpec((BLK, HW), lambda i: (i, 0)),
            pl.BlockSpec((BLK, HW), lambda i: (i, 0)),
            pl.BlockSpec((BLK, HW), lambda i: (i, 0)),
            pl.BlockSpec((HID, D_IN), lambda i: (0, 0)),
            pl.BlockSpec((1, D_IN), lambda i: (0, 0)),
        ],
        out_specs=pl.BlockSpec((BLK, D_IN), lambda i: (i, 0)),
        out_shape=jax.ShapeDtypeStruct((N_NODES, D_IN), jnp.float32),
    )(V, H1, H2, HC, W2, b2row)


def kernel(V, edge_index, rel_ids, W1, b1, W2, b2):
    A_lo, A_hi = _stage_a(V, W1)
    B_lo, B_hi = _stage_b(V, W1, b1.reshape(1, HID))
    B_lo = B_lo.reshape(NUM_REL * N_NODES, HW)
    B_hi = B_hi.reshape(NUM_REL * N_NODES, HW)
    src = edge_index[0]
    dst = edge_index[1]
    Hh1 = _sc_edges(A_lo, B_lo, src, dst, rel_ids)
    Hh2 = _sc_edges(A_hi, B_hi, src, dst, rel_ids)
    Hhc = _sc_count(dst)
    H1 = jnp.concatenate([Hh1[:HALF], Hh1[PAD_HALF:PAD_HALF + HALF]], axis=0)
    H2 = jnp.concatenate([Hh2[:HALF], Hh2[PAD_HALF:PAD_HALF + HALF]], axis=0)
    HC = jnp.concatenate([Hhc[:HALF], Hhc[PAD_HALF:PAD_HALF + HALF]], axis=0)
    return _stage_out(V, H1, H2, HC, W2, b2.reshape(1, D_IN))


# speedup vs baseline: 1.7663x; 1.2140x over previous
import jax
import jax.numpy as jnp
from jax import lax
from jax.experimental import pallas as pl
from jax.experimental.pallas import tpu as pltpu
from jax.experimental.pallas import tpu_sc as plsc

N_NODES = 10000
E_TOT = 320000
D_IN = 128
NUM_REL = 16
HID = 256
HW = 128
HALF = N_NODES // 2
PAD_HALF = 5120
ROWS_PER_TILE = PAD_HALF // 16
N_TILE_EDGES = E_TOT // 16
CHUNK = 128
N_FULL = N_TILE_EDGES // CHUNK
TAIL = N_TILE_EDGES - N_FULL * CHUNK
TILE_BLOCKS = 157
N_BLOCKS = 2512
NB = 10
BLK = N_NODES // NB


def _mm_a_body(v_ref, w1_ref, alo_ref, ahi_ref):
    alo_ref[...] = jnp.dot(v_ref[...], w1_ref[:D_IN, :HW],
                           preferred_element_type=jnp.float32)
    ahi_ref[...] = jnp.dot(v_ref[...], w1_ref[:D_IN, HW:],
                           preferred_element_type=jnp.float32)


def _stage_a(V, W1):
    return pl.pallas_call(
        _mm_a_body,
        grid=(NB,),
        in_specs=[
            pl.BlockSpec((BLK, D_IN), lambda i: (i, 0)),
            pl.BlockSpec((2 * D_IN + NUM_REL, HID), lambda i: (0, 0)),
        ],
        out_specs=[
            pl.BlockSpec((BLK, HW), lambda i: (i, 0)),
            pl.BlockSpec((BLK, HW), lambda i: (i, 0)),
        ],
        out_shape=[
            jax.ShapeDtypeStruct((N_NODES, HW), jnp.float32),
            jax.ShapeDtypeStruct((N_NODES, HW), jnp.float32),
        ],
    )(V, W1)


def _mm_b_body(v_ref, w1_ref, b1_ref, blo_ref, bhi_ref):
    r = pl.program_id(0)
    t = w1_ref[pl.ds(2 * D_IN + r, 1), :] + b1_ref[...]
    mm_lo = jnp.dot(v_ref[...], w1_ref[D_IN:2 * D_IN, :HW],
                    preferred_element_type=jnp.float32)
    mm_hi = jnp.dot(v_ref[...], w1_ref[D_IN:2 * D_IN, HW:],
                    preferred_element_type=jnp.float32)
    blo_ref[0] = mm_lo + t[:, :HW]
    bhi_ref[0] = mm_hi + t[:, HW:]


def _stage_b(V, W1, b1row):
    return pl.pallas_call(
        _mm_b_body,
        grid=(NUM_REL, NB),
        in_specs=[
            pl.BlockSpec((BLK, D_IN), lambda r, i: (i, 0)),
            pl.BlockSpec((2 * D_IN + NUM_REL, HID), lambda r, i: (0, 0)),
            pl.BlockSpec((1, HID), lambda r, i: (0, 0)),
        ],
        out_specs=[
            pl.BlockSpec((1, BLK, HW), lambda r, i: (r, i, 0)),
            pl.BlockSpec((1, BLK, HW), lambda r, i: (r, i, 0)),
        ],
        out_shape=[
            jax.ShapeDtypeStruct((NUM_REL, N_NODES, HW), jnp.float32),
            jax.ShapeDtypeStruct((NUM_REL, N_NODES, HW), jnp.float32),
        ],
    )(V, W1, b1row)


def _idx_body(ei_ref, rel_ref, out_ref):
    src = ei_ref[0]
    dst = ei_ref[1]
    out_ref[:, 0, :] = jnp.where(dst < 0, 0, src)
    out_ref[:, 1, :] = dst
    out_ref[:, 2, :] = jnp.where(dst < 0, 0, rel_ref[...] * N_NODES + dst)


def _stage_idx(ei_pad, rel_pad):
    return pl.pallas_call(
        _idx_body,
        grid=(1,),
        in_specs=[
            pl.BlockSpec((2, N_BLOCKS, CHUNK), lambda i: (0, 0, 0)),
            pl.BlockSpec((N_BLOCKS, CHUNK), lambda i: (0, 0)),
        ],
        out_specs=pl.BlockSpec((N_BLOCKS, 3, CHUNK), lambda i: (0, 0, 0)),
        out_shape=jax.ShapeDtypeStruct((N_BLOCKS, 3, CHUNK), jnp.int32),
    )(ei_pad, rel_pad)


def _sc_gather_body(a_hbm, b2_hbm, i3_hbm, h_out_hbm,
                    hb0, hb1, ib0, ib1, iv0, iv1, ivp, h_shared,
                    sA0, sA1, sB0, sB1, sS0, sS1):
    c = lax.axis_index("c")
    s = lax.axis_index("s")
    nbase = c * HALF
    bbase = s * TILE_BLOCKS
    iot = lax.iota(jnp.int32, 16)
    zero16 = jnp.zeros((16,), jnp.float32)
    stripe = s * ROWS_PER_TILE

    def _zbody(e, carry):
        for j in range(HW // 16):
            hb0[e, pl.ds(16 * j, 16)] = zero16
            hb1[e, pl.ds(16 * j, 16)] = zero16
        return carry

    lax.fori_loop(0, CHUNK, _zbody, 0)

    def _tbody(j, carry):
        ivp[pl.ds(16 * j, 16)] = HALF + iot
        return carry

    lax.fori_loop(0, CHUNK // 16, _tbody, 0)

    pltpu.sync_copy(hb0, h_shared.at[pl.ds(stripe, CHUNK)])
    pltpu.sync_copy(hb0, h_shared.at[pl.ds(stripe + CHUNK, CHUNK)])
    pltpu.sync_copy(hb0.at[pl.ds(0, 64)],
                    h_shared.at[pl.ds(stripe + 2 * CHUNK, 64)])
    plsc.subcore_barrier()

    def _load_idx(k, ib):
        pltpu.sync_copy(i3_hbm.at[bbase + k], ib)

    def _owned_idx(ib, iv):
        def _ib(j, carry):
            dd = ib[1, pl.ds(16 * j, 16)]
            owned = (dd >= nbase) & (dd < nbase + HALF)
            iv[pl.ds(16 * j, 16)] = jnp.where(owned, dd - nbase, HALF + iot)
            return carry

        lax.fori_loop(0, CHUNK // 16, _ib, 0)

    def _relu(hb):
        def _eb(e, carry):
            for j in range(HW // 16):
                hb[e, pl.ds(16 * j, 16)] = jnp.maximum(
                    hb[e, pl.ds(16 * j, 16)], 0.0)
            return carry

        lax.fori_loop(0, CHUNK, _eb, 0)

    def _gA(ib, hb, sem):
        return pltpu.make_async_copy(a_hbm.at[ib.at[0]], hb, sem)

    def _gB_start(ib, hb, sem):
        return pltpu.async_copy(b2_hbm.at[ib.at[2]], hb, sem, add=True)

    def _sc_start(hb, iv, sem):
        pltpu.async_copy(hb, h_shared.at[iv], sem, add=True)

    def _sc_wait(hb, iv, sem):
        pltpu.make_async_copy(hb, h_shared.at[iv], sem).wait()

    _sc_start(hb0, ivp, sS0)
    _sc_start(hb1, ivp, sS1)
    _load_idx(0, ib0)
    _owned_idx(ib0, iv0)
    _sc_wait(hb0, ivp, sS0)
    _gA(ib0, hb0, sA0).start()

    def _pair(i, carry):
        b = 2 * i + 1
        _sc_wait(hb1, iv1, sS1)
        _load_idx(b, ib1)
        _owned_idx(ib1, iv1)
        _gA(ib0, hb0, sA0).wait()
        b0 = _gB_start(ib0, hb0, sB0)
        _gA(ib1, hb1, sA1).start()
        b0.wait()
        _relu(hb0)
        _sc_start(hb0, iv0, sS0)
        _gA(ib1, hb1, sA1).wait()
        _gB_start(ib1, hb1, sB1).wait()
        _relu(hb1)
        _sc_start(hb1, iv1, sS1)
        _sc_wait(hb0, iv0, sS0)
        _load_idx(b + 1, ib0)
        _owned_idx(ib0, iv0)
        _gA(ib0, hb0, sA0).start()
        return carry

    lax.fori_loop(0, TILE_BLOCKS // 2, _pair, 0)

    _gA(ib0, hb0, sA0).wait()
    _gB_start(ib0, hb0, sB0).wait()
    _relu(hb0)
    pltpu.sync_copy(hb0, h_shared.at[iv0], add=True)
    _sc_wait(hb1, iv1, sS1)

    plsc.subcore_barrier()
    out_base = c * PAD_HALF + stripe
    for k in range(2):
        pltpu.sync_copy(h_shared.at[pl.ds(stripe + k * CHUNK, CHUNK)], hb0)
        pltpu.sync_copy(hb0, h_out_hbm.at[pl.ds(out_base + k * CHUNK, CHUNK)])
    pltpu.sync_copy(h_shared.at[pl.ds(stripe + 2 * CHUNK, 64)],
                    hb0.at[pl.ds(0, 64)])
    pltpu.sync_copy(hb0.at[pl.ds(0, 64)],
                    h_out_hbm.at[pl.ds(out_base + 2 * CHUNK, 64)])


def _sc_edges(A, B2, I3):
    mesh = plsc.VectorSubcoreMesh(core_axis_name="c", subcore_axis_name="s")
    f = pl.kernel(
        _sc_gather_body,
        mesh=mesh,
        out_type=jax.ShapeDtypeStruct((2 * PAD_HALF, HW), jnp.float32),
        scratch_types=[
            pltpu.VMEM((CHUNK, HW), jnp.float32),
            pltpu.VMEM((CHUNK, HW), jnp.float32),
            pltpu.VMEM((3, CHUNK), jnp.int32),
            pltpu.VMEM((3, CHUNK), jnp.int32),
            pltpu.VMEM((CHUNK,), jnp.int32),
            pltpu.VMEM((CHUNK,), jnp.int32),
            pltpu.VMEM((CHUNK,), jnp.int32),
            pltpu.VMEM_SHARED((PAD_HALF, HW), jnp.float32),
            pltpu.SemaphoreType.DMA,
            pltpu.SemaphoreType.DMA,
            pltpu.SemaphoreType.DMA,
            pltpu.SemaphoreType.DMA,
            pltpu.SemaphoreType.DMA,
            pltpu.SemaphoreType.DMA,
        ],
    )
    return f(A, B2, I3)


def _sc_count_body(dst_hbm, h_out_hbm,
                   hbuf, dvec, ivec, dvec_t, ivec_t, h_shared):
    c = lax.axis_index("c")
    s = lax.axis_index("s")
    nbase = c * HALF
    ebase = s * N_TILE_EDGES
    iot = lax.iota(jnp.int32, 16)
    zero16 = jnp.zeros((16,), jnp.float32)
    stripe = s * ROWS_PER_TILE

    def _zbody(e, carry):
        for j in range(HW // 16):
            hbuf[e, pl.ds(16 * j, 16)] = zero16
        return carry

    lax.fori_loop(0, CHUNK, _zbody, 0)
    pltpu.sync_copy(hbuf, h_shared.at[pl.ds(stripe, CHUNK)])
    pltpu.sync_copy(hbuf, h_shared.at[pl.ds(stripe + CHUNK, CHUNK)])
    pltpu.sync_copy(hbuf.at[pl.ds(0, 64)],
                    h_shared.at[pl.ds(stripe + 2 * CHUNK, 64)])
    plsc.subcore_barrier()

    one0 = jnp.where(iot == 0, 1.0, 0.0).astype(jnp.float32)

    def _fill_body(e, carry):
        hbuf[e, pl.ds(0, 16)] = one0
        return carry

    lax.fori_loop(0, CHUNK, _fill_body, 0)

    def _oidx(dv, iv, n_e):
        def _ib(j, carry):
            dd = dv[pl.ds(16 * j, 16)]
            owned = (dd >= nbase) & (dd < nbase + HALF)
            iv[pl.ds(16 * j, 16)] = jnp.where(owned, dd - nbase, HALF + iot)
            return carry

        lax.fori_loop(0, n_e // 16, _ib, 0)

    def _process(cbase, n_e, dv, iv):
        pltpu.sync_copy(dst_hbm.at[pl.ds(cbase, n_e)], dv)
        _oidx(dv, iv, n_e)
        h_src = hbuf if n_e == CHUNK else hbuf.at[pl.ds(0, n_e)]
        pltpu.sync_copy(h_src, h_shared.at[iv], add=True)

    def _chunk_body(i, carry):
        _process(ebase + i * CHUNK, CHUNK, dvec, ivec)
        return carry

    lax.fori_loop(0, N_FULL, _chunk_body, 0)
    _process(ebase + N_FULL * CHUNK, TAIL, dvec_t, ivec_t)

    plsc.subcore_barrier()
    out_base = c * PAD_HALF + stripe
    for k in range(2):
        pltpu.sync_copy(h_shared.at[pl.ds(stripe + k * CHUNK, CHUNK)], hbuf)
        pltpu.sync_copy(hbuf, h_out_hbm.at[pl.ds(out_base + k * CHUNK, CHUNK)])
    pltpu.sync_copy(h_shared.at[pl.ds(stripe + 2 * CHUNK, 64)],
                    hbuf.at[pl.ds(0, 64)])
    pltpu.sync_copy(hbuf.at[pl.ds(0, 64)],
                    h_out_hbm.at[pl.ds(out_base + 2 * CHUNK, 64)])


def _sc_count(dst):
    mesh = plsc.VectorSubcoreMesh(core_axis_name="c", subcore_axis_name="s")
    f = pl.kernel(
        _sc_count_body,
        mesh=mesh,
        out_type=jax.ShapeDtypeStruct((2 * PAD_HALF, HW), jnp.float32),
        scratch_types=[
            pltpu.VMEM((CHUNK, HW), jnp.float32),
            pltpu.VMEM((CHUNK,), jnp.int32),
            pltpu.VMEM((CHUNK,), jnp.int32),
            pltpu.VMEM((TAIL,), jnp.int32),
            pltpu.VMEM((TAIL,), jnp.int32),
            pltpu.VMEM_SHARED((PAD_HALF, HW), jnp.float32),
        ],
    )
    return f(dst)


def _mm_out_body(v_ref, h1_ref, h2_ref, hc_ref, w2_ref, b2_ref, out_ref):
    mm = jnp.dot(h1_ref[...], w2_ref[:HW, :],
                 preferred_element_type=jnp.float32)
    mm += jnp.dot(h2_ref[...], w2_ref[HW:, :],
                  preferred_element_type=jnp.float32)
    cnt = hc_ref[:, 0:1]
    out_ref[...] = v_ref[...] + mm + cnt * b2_ref[...]


def _stage_out(V, H1, H2, HC, W2, b2row):
    return pl.pallas_call(
        _mm_out_body,
        grid=(NB,),
        in_specs=[
            pl.BlockSpec((BLK, D_IN), lambda i: (i, 0)),
            pl.BlockSpec((BLK, HW), lambda i: (i, 0)),
            pl.BlockSpec((BLK, HW), lambda i: (i, 0)),
            pl.BlockSpec((BLK, HW), lambda i: (i, 0)),
            pl.BlockSpec((HID, D_IN), lambda i: (0, 0)),
            pl.BlockSpec((1, D_IN), lambda i: (0, 0)),
        ],
        out_specs=pl.BlockSpec((BLK, D_IN), lambda i: (i, 0)),
        out_shape=jax.ShapeDtypeStruct((N_NODES, D_IN), jnp.float32),
    )(V, H1, H2, HC, W2, b2row)


def kernel(V, edge_index, rel_ids, W1, b1, W2, b2):
    A_lo, A_hi = _stage_a(V, W1)
    B_lo, B_hi = _stage_b(V, W1, b1.reshape(1, HID))
    B_lo = B_lo.reshape(NUM_REL * N_NODES, HW)
    B_hi = B_hi.reshape(NUM_REL * N_NODES, HW)
    pad = N_BLOCKS * CHUNK - E_TOT
    src_pad = jnp.pad(edge_index[0], (0, pad))
    dst_pad = jnp.pad(edge_index[1], (0, pad), constant_values=-1)
    ei_pad = jnp.stack([src_pad, dst_pad]).reshape(2, N_BLOCKS, CHUNK)
    rel_pad = jnp.pad(rel_ids, (0, pad)).reshape(N_BLOCKS, CHUNK)
    I3 = _stage_idx(ei_pad, rel_pad)
    dst = edge_index[1]
    Hh1 = _sc_edges(A_lo, B_lo, I3)
    Hh2 = _sc_edges(A_hi, B_hi, I3)
    Hhc = _sc_count(dst)
    H1 = jnp.concatenate([Hh1[:HALF], Hh1[PAD_HALF:PAD_HALF + HALF]], axis=0)
    H2 = jnp.concatenate([Hh2[:HALF], Hh2[PAD_HALF:PAD_HALF + HALF]], axis=0)
    HC = jnp.concatenate([Hhc[:HALF], Hhc[PAD_HALF:PAD_HALF + HALF]], axis=0)
    return _stage_out(V, H1, H2, HC, W2, b2.reshape(1, D_IN))
